# baseline (speedup 1.0000x reference)
import jax
import jax.numpy as jnp
from jax.experimental import pallas as pl
from jax.experimental.pallas import tpu as pltpu

_POOL = 7
_SCALE = 1.0 / 16
_K = 5


def _bin_ranges(lo, hi_incl, dim):
    length = jnp.clip(hi_incl + 1, 0, dim) - lo
    length = jnp.maximum(length, 1)
    i = jnp.arange(_POOL, dtype=jnp.int32)
    start = lo[:, None] + (i[None, :] * length[:, None]) // _POOL
    end = lo[:, None] + ((i[None, :] + 1) * length[:, None] + _POOL - 1) // _POOL
    start = jnp.clip(start, 0, dim - 1)
    end = jnp.clip(end, start + 1, dim)
    return start, end


def _rmq_indices(start, end, dim):
    seg = end - start
    lvl = jnp.where(seg >= 4, 2, jnp.where(seg >= 2, 1, 0))
    p2 = jnp.int32(1) << lvl
    a = lvl * dim + start
    b = lvl * dim + (end - p2)
    return a.astype(jnp.int32), b.astype(jnp.int32)


def _clamped_indices(start, end):
    k = jnp.arange(_K, dtype=jnp.int32)
    idx = start[:, :, None] + jnp.minimum(k[None, None, :], (end - start - 1)[:, :, None])
    return idx.astype(jnp.int32)


_B = 16


def _roi_kernel(idx_ref, tab_ref, perm_ref, out_ref, rbuf, xbuf):
    g = pl.program_id(0)

    for u in range(_B):
        b = g * _B + u
        r = rbuf.at[u % 2]
        x = xbuf.at[u % 2]

        for i in range(_POOL):
            r[i] = jnp.maximum(tab_ref[idx_ref[i, 0, b]], tab_ref[idx_ref[i, 1, b]])

        for j in range(_POOL):
            acc = r[:, pl.ds(idx_ref[j, 2, b], 1), :]
            for k in range(1, _K):
                acc = jnp.maximum(acc, r[:, pl.ds(idx_ref[j, 2 + k, b], 1), :])
            x[j * _POOL : (j + 1) * _POOL, :] = acc[:, 0, :]

        out_ref[u] = jax.lax.dot_general(
            x[...],
            perm_ref[...],
            dimension_numbers=(((0,), (0,)), ((), ())),
            preferred_element_type=jnp.float32,
        )


def kernel(img_features, roi_boxes):
    feat = jnp.transpose(img_features[0], (1, 2, 0))
    H, W, C = feat.shape
    N = roi_boxes.shape[0]

    t1 = jnp.maximum(feat, jnp.concatenate([feat[1:], feat[-1:]], axis=0))
    t2 = jnp.maximum(t1, jnp.concatenate([t1[2:], t1[-2:]], axis=0))
    tab = jnp.concatenate([feat, t1, t2], axis=0)

    boxes = jnp.floor(roi_boxes.astype(jnp.float32) * _SCALE).astype(jnp.int32)
    rstart, rend = _bin_ranges(boxes[:, 1], boxes[:, 3], H)
    cstart, cend = _bin_ranges(boxes[:, 0], boxes[:, 2], W)
    ra, rb = _rmq_indices(rstart, rend, H)
    cidx = _clamped_indices(cstart, cend)
    idxs = jnp.concatenate(
        [
            jnp.stack([jnp.transpose(ra), jnp.transpose(rb)], axis=1),
            jnp.transpose(cidx, (1, 2, 0)),
        ],
        axis=1,
    )

    s = jnp.arange(_POOL * _POOL, dtype=jnp.int32)
    dst = (s % _POOL) * _POOL + s // _POOL
    perm = jax.nn.one_hot(dst, _POOL * _POOL, dtype=jnp.float32)

    out = pl.pallas_call(
        _roi_kernel,
        out_shape=jax.ShapeDtypeStruct((N, C, _POOL * _POOL), feat.dtype),
        grid_spec=pltpu.PrefetchScalarGridSpec(
            num_scalar_prefetch=1,
            grid=(N // _B,),
            in_specs=[
                pl.BlockSpec((3 * H, W, C), lambda b, idx_ref: (0, 0, 0)),
                pl.BlockSpec((_POOL * _POOL, _POOL * _POOL), lambda b, idx_ref: (0, 0)),
            ],
            out_specs=pl.BlockSpec(
                (_B, C, _POOL * _POOL), lambda b, idx_ref: (b, 0, 0)
            ),
            scratch_shapes=[
                pltpu.VMEM((2, _POOL, W, C), feat.dtype),
                pltpu.VMEM((2, _POOL * _POOL, C), feat.dtype),
            ],
        ),
        compiler_params=pltpu.CompilerParams(
            dimension_semantics=("parallel",),
        ),
        name="roi_maxpool",
    )(idxs, tab, perm)

    return out.reshape(N, C, _POOL, _POOL)


# speedup vs baseline: 1.2023x; 1.2023x over previous
import jax
import jax.numpy as jnp
from jax.experimental import pallas as pl
from jax.experimental.pallas import tpu as pltpu

_POOL = 7
_SCALE = 1.0 / 16
_K = 5


def _bin_ranges(lo, hi_incl, dim):
    length = jnp.clip(hi_incl + 1, 0, dim) - lo
    length = jnp.maximum(length, 1)
    i = jnp.arange(_POOL, dtype=jnp.int32)
    start = lo[:, None] + (i[None, :] * length[:, None]) // _POOL
    end = lo[:, None] + ((i[None, :] + 1) * length[:, None] + _POOL - 1) // _POOL
    start = jnp.clip(start, 0, dim - 1)
    end = jnp.clip(end, start + 1, dim)
    return start, end


def _len_table_indices(start, end, dim):
    seg = jnp.clip(end - start, 1, _K)
    return ((seg - 1) * dim + start).astype(jnp.int32)


def _clamped_indices(start, end):
    k = jnp.arange(_K, dtype=jnp.int32)
    idx = start[:, :, None] + jnp.minimum(k[None, None, :], (end - start - 1)[:, :, None])
    return idx.astype(jnp.int32)


_B = 16


def _roi_kernel(idx_ref, tab_ref, out_ref, rbuf):
    g = pl.program_id(0)

    for u in range(_B):
        b = g * _B + u
        r = rbuf.at[u % 2]

        for i in range(_POOL):
            r[i] = tab_ref[idx_ref[i, 0, b]]

        for j in range(_POOL):
            acc = r[:, pl.ds(idx_ref[j, 2, b], 1), :]
            for k in range(1, _K):
                acc = jnp.maximum(acc, r[:, pl.ds(idx_ref[j, 2 + k, b], 1), :])
            out_ref[u, j, :, :] = acc[:, 0, :]


def kernel(img_features, roi_boxes):
    feat = jnp.transpose(img_features[0], (1, 2, 0))
    H, W, C = feat.shape
    N = roi_boxes.shape[0]

    def _sh(x, d):
        return jnp.concatenate([x[d:], x[-1:].repeat(d, axis=0)], axis=0)

    t1 = feat
    t2 = jnp.maximum(t1, _sh(t1, 1))
    t3 = jnp.maximum(t2, _sh(t1, 2))
    t4 = jnp.maximum(t2, _sh(t2, 2))
    t5 = jnp.maximum(t4, _sh(t1, 4))
    tab = jnp.concatenate([t1, t2, t3, t4, t5], axis=0)

    boxes = jnp.floor(roi_boxes.astype(jnp.float32) * _SCALE).astype(jnp.int32)
    rstart, rend = _bin_ranges(boxes[:, 1], boxes[:, 3], H)
    cstart, cend = _bin_ranges(boxes[:, 0], boxes[:, 2], W)
    ra = _len_table_indices(rstart, rend, H)
    cidx = _clamped_indices(cstart, cend)
    idxs = jnp.concatenate(
        [
            jnp.stack([jnp.transpose(ra), jnp.transpose(ra)], axis=1),
            jnp.transpose(cidx, (1, 2, 0)),
        ],
        axis=1,
    )

    out = pl.pallas_call(
        _roi_kernel,
        out_shape=jax.ShapeDtypeStruct((N, _POOL, _POOL, C), feat.dtype),
        grid_spec=pltpu.PrefetchScalarGridSpec(
            num_scalar_prefetch=1,
            grid=(N // _B,),
            in_specs=[
                pl.BlockSpec((_K * H, W, C), lambda b, idx_ref: (0, 0, 0)),
            ],
            out_specs=pl.BlockSpec(
                (_B, _POOL, _POOL, C), lambda b, idx_ref: (b, 0, 0, 0)
            ),
            scratch_shapes=[pltpu.VMEM((2, _POOL, W, C), feat.dtype)],
        ),
        compiler_params=pltpu.CompilerParams(
            dimension_semantics=("parallel",),
        ),
        name="roi_maxpool",
    )(idxs, tab)

    return jnp.transpose(out, (0, 3, 2, 1))


# speedup vs baseline: 1.2740x; 1.0597x over previous
import jax
import jax.numpy as jnp
from jax.experimental import pallas as pl
from jax.experimental.pallas import tpu as pltpu

_POOL = 7
_SCALE = 1.0 / 16
_K = 5


def _bin_ranges(lo, hi_incl, dim):
    length = jnp.clip(hi_incl + 1, 0, dim) - lo
    length = jnp.maximum(length, 1)
    i = jnp.arange(_POOL, dtype=jnp.int32)
    start = lo[:, None] + (i[None, :] * length[:, None]) // _POOL
    end = lo[:, None] + ((i[None, :] + 1) * length[:, None] + _POOL - 1) // _POOL
    start = jnp.clip(start, 0, dim - 1)
    end = jnp.clip(end, start + 1, dim)
    return start, end


def _len_table_indices(start, end, dim):
    seg = jnp.clip(end - start, 1, _K)
    return ((seg - 1) * dim + start).astype(jnp.int32)


def _clamped_indices(start, end):
    k = jnp.arange(_K, dtype=jnp.int32)
    idx = start[:, :, None] + jnp.minimum(k[None, None, :], (end - start - 1)[:, :, None])
    return idx.astype(jnp.int32)


_B = 32


def _roi_kernel(idx_ref, tab_ref, out_ref, rbuf):
    g = pl.program_id(0)

    for u in range(_B):
        b = g * _B + u
        r = rbuf.at[u % 2]

        for i in range(_POOL):
            r[i] = tab_ref[idx_ref[i, 0, b]]

        for j in range(_POOL):
            acc = r[:, pl.ds(idx_ref[j, 2, b], 1), :]
            for k in range(1, _K):
                acc = jnp.maximum(acc, r[:, pl.ds(idx_ref[j, 2 + k, b], 1), :])
            out_ref[u, j, :, :] = acc[:, 0, :]


def kernel(img_features, roi_boxes):
    feat = jnp.transpose(img_features[0], (1, 2, 0))
    H, W, C = feat.shape
    N = roi_boxes.shape[0]

    def _sh(x, d):
        return jnp.concatenate([x[d:], x[-1:].repeat(d, axis=0)], axis=0)

    t1 = feat
    t2 = jnp.maximum(t1, _sh(t1, 1))
    t3 = jnp.maximum(t2, _sh(t1, 2))
    t4 = jnp.maximum(t2, _sh(t2, 2))
    t5 = jnp.maximum(t4, _sh(t1, 4))
    tab = jnp.concatenate([t1, t2, t3, t4, t5], axis=0)

    boxes = jnp.floor(roi_boxes.astype(jnp.float32) * _SCALE).astype(jnp.int32)
    rstart, rend = _bin_ranges(boxes[:, 1], boxes[:, 3], H)
    cstart, cend = _bin_ranges(boxes[:, 0], boxes[:, 2], W)
    ra = _len_table_indices(rstart, rend, H)
    cidx = _clamped_indices(cstart, cend)
    idxs = jnp.concatenate(
        [
            jnp.stack([jnp.transpose(ra), jnp.transpose(ra)], axis=1),
            jnp.transpose(cidx, (1, 2, 0)),
        ],
        axis=1,
    )

    out = pl.pallas_call(
        _roi_kernel,
        out_shape=jax.ShapeDtypeStruct((N, _POOL, _POOL, C), feat.dtype),
        grid_spec=pltpu.PrefetchScalarGridSpec(
            num_scalar_prefetch=1,
            grid=(N // _B,),
            in_specs=[
                pl.BlockSpec((_K * H, W, C), lambda b, idx_ref: (0, 0, 0)),
            ],
            out_specs=pl.BlockSpec(
                (_B, _POOL, _POOL, C), lambda b, idx_ref: (b, 0, 0, 0)
            ),
            scratch_shapes=[pltpu.VMEM((2, _POOL, W, C), feat.dtype)],
        ),
        compiler_params=pltpu.CompilerParams(
            dimension_semantics=("parallel",),
        ),
        name="roi_maxpool",
    )(idxs, tab)

    return jnp.transpose(out, (0, 3, 2, 1))
